# baseline (speedup 1.0000x reference)
import functools

import jax
import jax.numpy as jnp
from jax import lax
from jax.experimental import pallas as pl
from jax.experimental.pallas import tpu as pltpu
from jax.experimental.pallas import tpu_sc as plsc

_TAU = 0.5
_WEIGHT = 5.0
_K = 5
_N = 16384
_D = 128
_L = 16
_NC = 2
_NS = 16
_NW = _NC * _NS

_N_SC = 4096
_N_TC = _N - _N_SC
_TC_BLK = _N - _N_SC

_RPW = _N_SC // _NW
_NG = _RPW // _L
_NJ = _D // _L

_ORDER = [sum(((p >> i) & 1) * (8 >> i) for i in range(4)) for p in range(16)]
_S_FOR_LEVEL = (8, 4, 2, 1)
_PASSES = ((0, 1, 2, 3, 4),)


def _gather(v, idx):
    return v.at[idx].get(mode="promise_in_bounds")


def _sc_partials_body(feat_hbm, cent_hbm, tcp_hbm, out_hbm, feat_v, cent_v, acc_v):
    wid = lax.axis_index("s") * _NC + lax.axis_index("c")
    base = _N_TC + wid * _RPW
    pltpu.sync_copy(cent_hbm, cent_v)
    pltpu.sync_copy(feat_hbm.at[pl.ds(base, _RPW)], feat_v)

    lane = lax.broadcasted_iota(jnp.int32, (_L,), 0)
    perms = {s: lane ^ s for s in _S_FOR_LEVEL}
    masks = {s: (lane & s) == 0 for s in _S_FOR_LEVEL}

    def fold(v, s):
        return v + _gather(v, perms[s])

    def merge(a, b, level):
        s = _S_FOR_LEVEL[level]
        return jnp.where(masks[s], fold(a, s), _gather(fold(b, s), perms[s]))

    def splat_sum(v):
        for s in _S_FOR_LEVEL:
            v = fold(v, s)
        return v

    c2s = []
    for k in range(_K):
        ch = [cent_v[k, pl.ds(j * _L, _L)] for j in range(_NJ)]
        acc = ch[0] * ch[0]
        for j in range(1, _NJ):
            acc = acc + ch[j] * ch[j]
        c2s.append(splat_sum(acc))

    zvec = jnp.zeros((_L,), jnp.float32)
    ones = jnp.full((_L,), 1.0, jnp.float32)
    for r in range(_L):
        acc_v[r, :] = zvec

    def group_body(g, carry):
        base_row = g * _L
        gvecs = [None] * _K
        for ks in _PASSES:
            cch = {k: [cent_v[k, pl.ds(j * _L, _L)] for j in range(_NJ)]
                   for k in ks}
            slots = {k: {} for k in ks}
            for p in range(_L):
                row = base_row + _ORDER[p]
                chunks = [feat_v[row, pl.ds(j * _L, _L)] for j in range(_NJ)]
                for k in ks:
                    t = chunks[0] * cch[k][0]
                    for j in range(1, _NJ):
                        t = t + chunks[j] * cch[k][j]
                    level = 0
                    while level in slots[k]:
                        t = merge(slots[k].pop(level), t, level)
                        level += 1
                    slots[k][level] = t
            for k in ks:
                gvecs[k] = slots[k][4]

        best = c2s[0] - 2.0 * gvecs[0]
        pred = jnp.zeros((_L,), jnp.int32)
        for k in range(1, _K):
            dk = c2s[k] - 2.0 * gvecs[k]
            better = dk < best
            best = jnp.where(better, dk, best)
            pred = jnp.where(better, k, pred)
        for k in range(_K):
            e = jnp.exp(gvecs[k] * (1.0 / _TAU))
            mk = pred == k
            acc_v[k, :] = acc_v[k, :] + jnp.where(mk, e, 0.0)
            acc_v[_K + k, :] = acc_v[_K + k, :] + e
            acc_v[2 * _K + k, :] = acc_v[2 * _K + k, :] + jnp.where(mk, ones, zvec)
        return carry

    lax.fori_loop(0, _NG, group_body, jnp.int32(0))
    pltpu.sync_copy(acc_v, out_hbm.at[pl.ds(wid * _L, _L)])


_sc_partials = functools.partial(
    pl.kernel,
    out_type=jax.ShapeDtypeStruct((_NW * _L, _L), jnp.float32),
    mesh=plsc.VectorSubcoreMesh(core_axis_name="c", subcore_axis_name="s"),
    scratch_types=[
        pltpu.VMEM((_RPW, _D), jnp.float32),
        pltpu.VMEM((_K, _D), jnp.float32),
        pltpu.VMEM((_L, _L), jnp.float32),
    ],
)(_sc_partials_body)


def _tc_part_kernel(feat_ref, cent_ref, out_ref):
    feat = feat_ref[...]
    cent = cent_ref[...]
    gt = lax.dot_general(cent, feat, (((1,), (1,)), ((), ())),
                         preferred_element_type=jnp.float32)
    c2 = jnp.sum(cent * cent, axis=1, keepdims=True)
    dist = c2 - 2.0 * gt
    colmask = lax.broadcasted_iota(jnp.int32, (1, _N), 1) < _N_TC
    best = dist[0:1, :]
    pred = jnp.zeros((1, _N), jnp.int32)
    for k in range(1, _K):
        dk = dist[k:k + 1, :]
        b = dk < best
        best = jnp.where(b, dk, best)
        pred = jnp.where(b, k, pred)
    rowi = lax.broadcasted_iota(jnp.int32, (8, 128), 0)
    lanei = lax.broadcasted_iota(jnp.int32, (8, 128), 1)
    out = jnp.zeros((8, 128), jnp.float32)
    for k in range(_K):
        ek = jnp.exp(gt[k:k + 1, :] * (1.0 / _TAU))
        mk = (pred == k) & colmask
        posk = jnp.sum(jnp.where(mk, ek, 0.0))
        negk = jnp.sum(jnp.where(colmask, ek, 0.0))
        cntk = jnp.sum(jnp.where(mk, 1.0, 0.0))
        out = jnp.where((rowi == 0) & (lanei == k), posk, out)
        out = jnp.where((rowi == 1) & (lanei == k), negk, out)
        out = jnp.where((rowi == 2) & (lanei == k), cntk, out)
    out_ref[...] = out


def _tc_partials(feat, centroids):
    return pl.pallas_call(
        _tc_part_kernel,
        out_shape=jax.ShapeDtypeStruct((8, 128), jnp.float32),
        in_specs=[
            pl.BlockSpec((_N, _D), lambda: (0, 0)),
            pl.BlockSpec((_K, _D), lambda: (0, 0)),
        ],
        out_specs=pl.BlockSpec((8, 128), lambda: (0, 0)),
    )(feat, centroids)


def _finish_kernel(sc_ref, tc_ref, out_ref):
    x = sc_ref[...]
    rows = _NW * _L
    rs = jnp.sum(x, axis=1, keepdims=True)
    rowt = lax.broadcasted_iota(jnp.int32, (rows, _L), 0) % _L
    lanei = lax.broadcasted_iota(jnp.int32, (rows, _L), 1)
    tot = jnp.sum(jnp.where(lanei == rowt, rs, 0.0), axis=0, keepdims=True)
    tcp = tc_ref[...]
    pos = tot[:, 0:_K] + tcp[0:1, 0:_K]
    neg = tot[:, _K:2 * _K] + tcp[1:2, 0:_K]
    cnt = tot[:, 2 * _K:3 * _K] + tcp[2:3, 0:_K]
    posm = pos / jnp.maximum(cnt, 1.0)
    negm = neg / jnp.float32(_N)
    term = jnp.where((cnt > 0.0) & (cnt < jnp.float32(_N)),
                     jnp.log(posm / negm), 0.0)
    loss = -jnp.sum(term) / jnp.float32(_K) * jnp.float32(_WEIGHT)
    out_ref[...] = jnp.reshape(loss, (1, 1))


@jax.jit
def _run(feat, centroids):
    tc_part = _tc_partials(feat, centroids)
    sc_part = _sc_partials(feat, centroids, tc_part)
    out = pl.pallas_call(
        _finish_kernel,
        out_shape=jax.ShapeDtypeStruct((1, 1), jnp.float32),
        in_specs=[
            pl.BlockSpec((_NW * _L, _L), lambda: (0, 0)),
            pl.BlockSpec((8, 128), lambda: (0, 0)),
        ],
        out_specs=pl.BlockSpec((1, 1), lambda: (0, 0)),
    )(sc_part, tc_part)
    return out[0, 0]


def kernel(feat, centroids, epoch):
    del epoch
    return _run(feat, centroids)


# speedup vs baseline: 1.1376x; 1.1376x over previous
import functools

import jax
import jax.numpy as jnp
from jax import lax
from jax.experimental import pallas as pl
from jax.experimental.pallas import tpu as pltpu
from jax.experimental.pallas import tpu_sc as plsc

_TAU = 0.5
_WEIGHT = 5.0
_K = 5
_N = 16384
_D = 128
_L = 16
_NC = 2
_NS = 16
_NW = _NC * _NS

_N_SC = 2048
_N_TC = _N - _N_SC
_TC_BLK = _N - _N_SC

_RPW = _N_SC // _NW
_NG = _RPW // _L
_NJ = _D // _L

_ORDER = [sum(((p >> i) & 1) * (8 >> i) for i in range(4)) for p in range(16)]
_S_FOR_LEVEL = (8, 4, 2, 1)
_PASSES = ((0, 1, 2, 3, 4),)


def _gather(v, idx):
    return v.at[idx].get(mode="promise_in_bounds")


def _sc_partials_body(feat_hbm, cent_hbm, out_hbm, feat_v, cent_v, acc_v):
    wid = lax.axis_index("s") * _NC + lax.axis_index("c")
    base = _N_TC + wid * _RPW
    pltpu.sync_copy(cent_hbm, cent_v)
    pltpu.sync_copy(feat_hbm.at[pl.ds(base, _RPW)], feat_v)

    lane = lax.broadcasted_iota(jnp.int32, (_L,), 0)
    perms = {s: lane ^ s for s in _S_FOR_LEVEL}
    masks = {s: (lane & s) == 0 for s in _S_FOR_LEVEL}

    def fold(v, s):
        return v + _gather(v, perms[s])

    def merge(a, b, level):
        s = _S_FOR_LEVEL[level]
        return jnp.where(masks[s], fold(a, s), _gather(fold(b, s), perms[s]))

    def splat_sum(v):
        for s in _S_FOR_LEVEL:
            v = fold(v, s)
        return v

    c2s = []
    for k in range(_K):
        ch = [cent_v[k, pl.ds(j * _L, _L)] for j in range(_NJ)]
        acc = ch[0] * ch[0]
        for j in range(1, _NJ):
            acc = acc + ch[j] * ch[j]
        c2s.append(splat_sum(acc))

    zvec = jnp.zeros((_L,), jnp.float32)
    ones = jnp.full((_L,), 1.0, jnp.float32)
    for r in range(_L):
        acc_v[r, :] = zvec

    def group_body(g, carry):
        base_row = g * _L
        gvecs = [None] * _K
        for ks in _PASSES:
            cch = {k: [cent_v[k, pl.ds(j * _L, _L)] for j in range(_NJ)]
                   for k in ks}
            slots = {k: {} for k in ks}
            for p in range(_L):
                row = base_row + _ORDER[p]
                chunks = [feat_v[row, pl.ds(j * _L, _L)] for j in range(_NJ)]
                for k in ks:
                    t = chunks[0] * cch[k][0]
                    for j in range(1, _NJ):
                        t = t + chunks[j] * cch[k][j]
                    level = 0
                    while level in slots[k]:
                        t = merge(slots[k].pop(level), t, level)
                        level += 1
                    slots[k][level] = t
            for k in ks:
                gvecs[k] = slots[k][4]

        best = c2s[0] - 2.0 * gvecs[0]
        pred = jnp.zeros((_L,), jnp.int32)
        for k in range(1, _K):
            dk = c2s[k] - 2.0 * gvecs[k]
            better = dk < best
            best = jnp.where(better, dk, best)
            pred = jnp.where(better, k, pred)
        for k in range(_K):
            e = jnp.exp(gvecs[k] * (1.0 / _TAU))
            mk = pred == k
            acc_v[k, :] = acc_v[k, :] + jnp.where(mk, e, 0.0)
            acc_v[_K + k, :] = acc_v[_K + k, :] + e
            acc_v[2 * _K + k, :] = acc_v[2 * _K + k, :] + jnp.where(mk, ones, zvec)
        return carry

    lax.fori_loop(0, _NG, group_body, jnp.int32(0))
    pltpu.sync_copy(acc_v, out_hbm.at[pl.ds(wid * _L, _L)])


_sc_partials = functools.partial(
    pl.kernel,
    out_type=jax.ShapeDtypeStruct((_NW * _L, _L), jnp.float32),
    mesh=plsc.VectorSubcoreMesh(core_axis_name="c", subcore_axis_name="s"),
    scratch_types=[
        pltpu.VMEM((_RPW, _D), jnp.float32),
        pltpu.VMEM((_K, _D), jnp.float32),
        pltpu.VMEM((_L, _L), jnp.float32),
    ],
)(_sc_partials_body)


def _tc_part_kernel(feat_ref, cent_ref, out_ref):
    feat = feat_ref[...]
    cent = cent_ref[...]
    gt = lax.dot_general(cent, feat, (((1,), (1,)), ((), ())),
                         preferred_element_type=jnp.float32)
    c2 = jnp.sum(cent * cent, axis=1, keepdims=True)
    dist = c2 - 2.0 * gt
    colmask = lax.broadcasted_iota(jnp.int32, (1, _N), 1) < _N_TC
    best = dist[0:1, :]
    pred = jnp.zeros((1, _N), jnp.int32)
    for k in range(1, _K):
        dk = dist[k:k + 1, :]
        b = dk < best
        best = jnp.where(b, dk, best)
        pred = jnp.where(b, k, pred)
    rowi = lax.broadcasted_iota(jnp.int32, (8, 128), 0)
    lanei = lax.broadcasted_iota(jnp.int32, (8, 128), 1)
    out = jnp.zeros((8, 128), jnp.float32)
    for k in range(_K):
        ek = jnp.exp(gt[k:k + 1, :] * (1.0 / _TAU))
        mk = (pred == k) & colmask
        posk = jnp.sum(jnp.where(mk, ek, 0.0))
        negk = jnp.sum(jnp.where(colmask, ek, 0.0))
        cntk = jnp.sum(jnp.where(mk, 1.0, 0.0))
        out = jnp.where((rowi == 0) & (lanei == k), posk, out)
        out = jnp.where((rowi == 1) & (lanei == k), negk, out)
        out = jnp.where((rowi == 2) & (lanei == k), cntk, out)
    out_ref[...] = out


def _tc_partials(feat, centroids):
    return pl.pallas_call(
        _tc_part_kernel,
        out_shape=jax.ShapeDtypeStruct((8, 128), jnp.float32),
        in_specs=[
            pl.BlockSpec((_N, _D), lambda: (0, 0)),
            pl.BlockSpec((_K, _D), lambda: (0, 0)),
        ],
        out_specs=pl.BlockSpec((8, 128), lambda: (0, 0)),
    )(feat, centroids)


def _finish_kernel(sc_ref, tc_ref, out_ref):
    x = sc_ref[...]
    rows = _NW * _L
    rs = jnp.sum(x, axis=1, keepdims=True)
    rowt = lax.broadcasted_iota(jnp.int32, (rows, _L), 0) % _L
    lanei = lax.broadcasted_iota(jnp.int32, (rows, _L), 1)
    tot = jnp.sum(jnp.where(lanei == rowt, rs, 0.0), axis=0, keepdims=True)
    tcp = tc_ref[...]
    pos = tot[:, 0:_K] + tcp[0:1, 0:_K]
    neg = tot[:, _K:2 * _K] + tcp[1:2, 0:_K]
    cnt = tot[:, 2 * _K:3 * _K] + tcp[2:3, 0:_K]
    posm = pos / jnp.maximum(cnt, 1.0)
    negm = neg / jnp.float32(_N)
    term = jnp.where((cnt > 0.0) & (cnt < jnp.float32(_N)),
                     jnp.log(posm / negm), 0.0)
    loss = -jnp.sum(term) / jnp.float32(_K) * jnp.float32(_WEIGHT)
    out_ref[...] = jnp.reshape(loss, (1, 1))


@jax.jit
def _run(feat, centroids):
    sc_part = _sc_partials(feat, centroids)
    tc_part = _tc_partials(feat, centroids)
    out = pl.pallas_call(
        _finish_kernel,
        out_shape=jax.ShapeDtypeStruct((1, 1), jnp.float32),
        in_specs=[
            pl.BlockSpec((_NW * _L, _L), lambda: (0, 0)),
            pl.BlockSpec((8, 128), lambda: (0, 0)),
        ],
        out_specs=pl.BlockSpec((1, 1), lambda: (0, 0)),
    )(sc_part, tc_part)
    return out[0, 0]


def kernel(feat, centroids, epoch):
    del epoch
    return _run(feat, centroids)


# speedup vs baseline: 1.1777x; 1.0352x over previous
import functools

import jax
import jax.numpy as jnp
from jax import lax
from jax.experimental import pallas as pl
from jax.experimental.pallas import tpu as pltpu
from jax.experimental.pallas import tpu_sc as plsc

_TAU = 0.5
_WEIGHT = 5.0
_K = 5
_N = 16384
_D = 128
_L = 16
_NC = 2
_NS = 16
_NW = _NC * _NS

_N_SC = 1024
_N_TC = _N - _N_SC
_TC_BLK = _N - _N_SC

_RPW = _N_SC // _NW
_NG = _RPW // _L
_NJ = _D // _L

_ORDER = [sum(((p >> i) & 1) * (8 >> i) for i in range(4)) for p in range(16)]
_S_FOR_LEVEL = (8, 4, 2, 1)
_PASSES = ((0, 1, 2, 3, 4),)


def _gather(v, idx):
    return v.at[idx].get(mode="promise_in_bounds")


def _sc_partials_body(feat_hbm, cent_hbm, out_hbm, feat_v, cent_v, acc_v):
    wid = lax.axis_index("s") * _NC + lax.axis_index("c")
    base = _N_TC + wid * _RPW
    pltpu.sync_copy(cent_hbm, cent_v)
    pltpu.sync_copy(feat_hbm.at[pl.ds(base, _RPW)], feat_v)

    lane = lax.broadcasted_iota(jnp.int32, (_L,), 0)
    perms = {s: lane ^ s for s in _S_FOR_LEVEL}
    masks = {s: (lane & s) == 0 for s in _S_FOR_LEVEL}

    def fold(v, s):
        return v + _gather(v, perms[s])

    def merge(a, b, level):
        s = _S_FOR_LEVEL[level]
        return jnp.where(masks[s], fold(a, s), _gather(fold(b, s), perms[s]))

    def splat_sum(v):
        for s in _S_FOR_LEVEL:
            v = fold(v, s)
        return v

    c2s = []
    for k in range(_K):
        ch = [cent_v[k, pl.ds(j * _L, _L)] for j in range(_NJ)]
        acc = ch[0] * ch[0]
        for j in range(1, _NJ):
            acc = acc + ch[j] * ch[j]
        c2s.append(splat_sum(acc))

    zvec = jnp.zeros((_L,), jnp.float32)
    ones = jnp.full((_L,), 1.0, jnp.float32)
    for r in range(_L):
        acc_v[r, :] = zvec

    def group_body(g, carry):
        base_row = g * _L
        gvecs = [None] * _K
        for ks in _PASSES:
            cch = {k: [cent_v[k, pl.ds(j * _L, _L)] for j in range(_NJ)]
                   for k in ks}
            slots = {k: {} for k in ks}
            for p in range(_L):
                row = base_row + _ORDER[p]
                chunks = [feat_v[row, pl.ds(j * _L, _L)] for j in range(_NJ)]
                for k in ks:
                    t = chunks[0] * cch[k][0]
                    for j in range(1, _NJ):
                        t = t + chunks[j] * cch[k][j]
                    level = 0
                    while level in slots[k]:
                        t = merge(slots[k].pop(level), t, level)
                        level += 1
                    slots[k][level] = t
            for k in ks:
                gvecs[k] = slots[k][4]

        best = c2s[0] - 2.0 * gvecs[0]
        pred = jnp.zeros((_L,), jnp.int32)
        for k in range(1, _K):
            dk = c2s[k] - 2.0 * gvecs[k]
            better = dk < best
            best = jnp.where(better, dk, best)
            pred = jnp.where(better, k, pred)
        for k in range(_K):
            e = jnp.exp(gvecs[k] * (1.0 / _TAU))
            mk = pred == k
            acc_v[k, :] = acc_v[k, :] + jnp.where(mk, e, 0.0)
            acc_v[_K + k, :] = acc_v[_K + k, :] + e
            acc_v[2 * _K + k, :] = acc_v[2 * _K + k, :] + jnp.where(mk, ones, zvec)
        return carry

    lax.fori_loop(0, _NG, group_body, jnp.int32(0))
    pltpu.sync_copy(acc_v, out_hbm.at[pl.ds(wid * _L, _L)])


_sc_partials = functools.partial(
    pl.kernel,
    out_type=jax.ShapeDtypeStruct((_NW * _L, _L), jnp.float32),
    mesh=plsc.VectorSubcoreMesh(core_axis_name="c", subcore_axis_name="s"),
    scratch_types=[
        pltpu.VMEM((_RPW, _D), jnp.float32),
        pltpu.VMEM((_K, _D), jnp.float32),
        pltpu.VMEM((_L, _L), jnp.float32),
    ],
)(_sc_partials_body)


def _tc_part_kernel(feat_ref, cent_ref, out_ref):
    feat = feat_ref[...]
    cent = cent_ref[...]
    gt = lax.dot_general(cent, feat, (((1,), (1,)), ((), ())),
                         preferred_element_type=jnp.float32)
    c2 = jnp.sum(cent * cent, axis=1, keepdims=True)
    dist = c2 - 2.0 * gt
    colmask = lax.broadcasted_iota(jnp.int32, (1, _N), 1) < _N_TC
    best = dist[0:1, :]
    pred = jnp.zeros((1, _N), jnp.int32)
    for k in range(1, _K):
        dk = dist[k:k + 1, :]
        b = dk < best
        best = jnp.where(b, dk, best)
        pred = jnp.where(b, k, pred)
    rowi = lax.broadcasted_iota(jnp.int32, (8, 128), 0)
    lanei = lax.broadcasted_iota(jnp.int32, (8, 128), 1)
    out = jnp.zeros((8, 128), jnp.float32)
    for k in range(_K):
        ek = jnp.exp(gt[k:k + 1, :] * (1.0 / _TAU))
        mk = (pred == k) & colmask
        posk = jnp.sum(jnp.where(mk, ek, 0.0))
        negk = jnp.sum(jnp.where(colmask, ek, 0.0))
        cntk = jnp.sum(jnp.where(mk, 1.0, 0.0))
        out = jnp.where((rowi == 0) & (lanei == k), posk, out)
        out = jnp.where((rowi == 1) & (lanei == k), negk, out)
        out = jnp.where((rowi == 2) & (lanei == k), cntk, out)
    out_ref[...] = out


def _tc_partials(feat, centroids):
    return pl.pallas_call(
        _tc_part_kernel,
        out_shape=jax.ShapeDtypeStruct((8, 128), jnp.float32),
        in_specs=[
            pl.BlockSpec((_N, _D), lambda: (0, 0)),
            pl.BlockSpec((_K, _D), lambda: (0, 0)),
        ],
        out_specs=pl.BlockSpec((8, 128), lambda: (0, 0)),
    )(feat, centroids)


def _finish_kernel(sc_ref, tc_ref, out_ref):
    x = sc_ref[...]
    rows = _NW * _L
    rs = jnp.sum(x, axis=1, keepdims=True)
    rowt = lax.broadcasted_iota(jnp.int32, (rows, _L), 0) % _L
    lanei = lax.broadcasted_iota(jnp.int32, (rows, _L), 1)
    tot = jnp.sum(jnp.where(lanei == rowt, rs, 0.0), axis=0, keepdims=True)
    tcp = tc_ref[...]
    pos = tot[:, 0:_K] + tcp[0:1, 0:_K]
    neg = tot[:, _K:2 * _K] + tcp[1:2, 0:_K]
    cnt = tot[:, 2 * _K:3 * _K] + tcp[2:3, 0:_K]
    posm = pos / jnp.maximum(cnt, 1.0)
    negm = neg / jnp.float32(_N)
    term = jnp.where((cnt > 0.0) & (cnt < jnp.float32(_N)),
                     jnp.log(posm / negm), 0.0)
    loss = -jnp.sum(term) / jnp.float32(_K) * jnp.float32(_WEIGHT)
    out_ref[...] = jnp.reshape(loss, (1, 1))


@jax.jit
def _run(feat, centroids):
    sc_part = _sc_partials(feat, centroids)
    tc_part = _tc_partials(feat, centroids)
    out = pl.pallas_call(
        _finish_kernel,
        out_shape=jax.ShapeDtypeStruct((1, 1), jnp.float32),
        in_specs=[
            pl.BlockSpec((_NW * _L, _L), lambda: (0, 0)),
            pl.BlockSpec((8, 128), lambda: (0, 0)),
        ],
        out_specs=pl.BlockSpec((1, 1), lambda: (0, 0)),
    )(sc_part, tc_part)
    return out[0, 0]


def kernel(feat, centroids, epoch):
    del epoch
    return _run(feat, centroids)
